# Initial kernel scaffold; baseline (speedup 1.0000x reference)
import jax
import jax.numpy as jnp
from jax.experimental import pallas as pl


def kernel(x, edge_index, W1, b1, W2, b2, thetas, fbias, Wf, bf, Wx, bx, Wc, bc):
    raise NotImplementedError("write your pallas kernel here")


# speedup vs baseline: 8.7186x; 8.7186x over previous
import functools

import jax
import jax.numpy as jnp
from jax import lax
from jax.experimental import pallas as pl
from jax.experimental.pallas import tpu as pltpu
from jax.experimental.pallas import tpu_sc as plsc

N = 10000
E = 320000
D_IN = 128
HID = 156
FN = 4
D_OUT = 2

NC = 2
NS = 16
NP = 10240
DP = 160
DPH = DP // 2
CH = 80
PER_TILE = E // NS
NCH = PER_TILE // CH
PER_TILE_DEG = E // (NC * NS)
NCH_DEG = PER_TILE_DEG // CH
RPT = NP // NS
WCH = 128
NWC = RPT // WCH

_mesh = plsc.VectorSubcoreMesh(
    core_axis_name="c", subcore_axis_name="s", num_cores=NC, num_subcores=NS)
_sc_params = pltpu.CompilerParams(use_tc_tiling_on_sc=False)


def _zero_fill(zbuf, rows, width):
    per_row = width // 16
    zv = jnp.zeros((16,), jnp.float32)

    def body(i, _):
        r = i // per_row
        cix = (i % per_row) * 16
        zbuf[r, pl.ds(cix, 16)] = zv
        return 0

    lax.fori_loop(0, rows * per_row, body, 0)


def _zero_shared(shared, zbuf, sidx):
    def zchunk(j, _):
        pltpu.sync_copy(zbuf, shared.at[pl.ds(sidx * RPT + j * WCH, WCH)])
        return 0

    lax.fori_loop(0, NWC, zchunk, 0)


def _drain_shared(shared, zbuf, out, sidx):
    def wchunk(j, _):
        start = sidx * RPT + j * WCH
        pltpu.sync_copy(shared.at[pl.ds(start, WCH)], zbuf)
        pltpu.sync_copy(zbuf, out.at[pl.ds(start, WCH)])
        return 0

    lax.fori_loop(0, NWC, wchunk, 0)


def _deg_body(rowi, out0, out1, idx_c, ones_buf, zbuf, shared):
    cidx = lax.axis_index("c")
    sidx = lax.axis_index("s")
    wid = sidx * NC + cidx

    lane = lax.broadcasted_iota(jnp.int32, (16,), 0)
    onehot = jnp.where(lane == 0, 1.0, 0.0).astype(jnp.float32)

    def fill(i, _):
        ones_buf[i, pl.ds(0, 16)] = onehot
        return 0

    lax.fori_loop(0, CH, fill, 0)
    _zero_fill(zbuf, WCH, 16)
    _zero_shared(shared, zbuf, sidx)
    plsc.subcore_barrier()

    def echunk(k, _):
        base = wid * PER_TILE_DEG + k * CH
        pltpu.sync_copy(rowi.at[pl.ds(base, CH)], idx_c)
        pltpu.sync_copy(ones_buf, shared.at[idx_c], add=True)
        return 0

    lax.fori_loop(0, NCH_DEG, echunk, 0)
    plsc.subcore_barrier()

    @pl.when(cidx == 0)
    def _():
        _drain_shared(shared, zbuf, out0, sidx)

    @pl.when(cidx == 1)
    def _():
        _drain_shared(shared, zbuf, out1, sidx)


_deg_kernel = functools.partial(
    pl.kernel,
    out_type=(
        jax.ShapeDtypeStruct((NP, 16), jnp.float32),
        jax.ShapeDtypeStruct((NP, 16), jnp.float32),
    ),
    mesh=_mesh,
    scratch_types=[
        pltpu.VMEM((CH,), jnp.int32),
        pltpu.VMEM((CH, 16), jnp.float32),
        pltpu.VMEM((WCH, 16), jnp.float32),
        pltpu.VMEM_SHARED((NP, 16), jnp.float32),
    ],
    compiler_params=_sc_params,
    name="deg_sc",
)(_deg_body)


def _spmm_half(table, out, rowi, coli,
               idx_r, idx_c, rows_buf, zbuf, shared, sem, sidx):
    _zero_fill(zbuf, WCH, DPH)
    _zero_shared(shared, zbuf, sidx)
    plsc.subcore_barrier()

    def echunk(k, _):
        base = sidx * PER_TILE + k * CH
        pltpu.sync_copy(rowi.at[pl.ds(base, CH)], idx_r)
        pltpu.async_copy(table.at[idx_r], rows_buf, sem).wait()
        pltpu.sync_copy(coli.at[pl.ds(base, CH)], idx_c)
        pltpu.sync_copy(rows_buf, shared.at[idx_c], add=True)
        return 0

    lax.fori_loop(0, NCH, echunk, 0)
    plsc.subcore_barrier()
    _drain_shared(shared, zbuf, out, sidx)


def _spmm_body(t_lo, t_hi, rowi, coli, out_lo, out_hi,
               idx_r, idx_c, rows_buf, zbuf, shared, sem):
    cidx = lax.axis_index("c")
    sidx = lax.axis_index("s")

    @pl.when(cidx == 0)
    def _():
        _spmm_half(t_lo, out_lo, rowi, coli,
                   idx_r, idx_c, rows_buf, zbuf, shared, sem, sidx)

    @pl.when(cidx == 1)
    def _():
        _spmm_half(t_hi, out_hi, rowi, coli,
                   idx_r, idx_c, rows_buf, zbuf, shared, sem, sidx)


def _make_spmm(nv):
    @functools.partial(
        pl.kernel,
        out_type=(
            jax.ShapeDtypeStruct((NP, DPH), jnp.float32),
            jax.ShapeDtypeStruct((NP, DPH), jnp.float32),
        ),
        mesh=_mesh,
        scratch_types=[
            pltpu.VMEM((CH,), jnp.int32),
            pltpu.VMEM((CH,), jnp.int32),
            pltpu.VMEM((CH, DPH), jnp.float32),
            pltpu.VMEM((WCH, DPH), jnp.float32),
            pltpu.VMEM_SHARED((NP, DPH), jnp.float32),
            pltpu.SemaphoreType.DMA,
        ],
        compiler_params=_sc_params,
        name=f"spmm_sc_{nv}",
    )
    def spmm(t_lo, t_hi, rowi, coli, out_lo, out_hi,
             idx_r, idx_c, rows_buf, zbuf, shared, sem):
        _spmm_body(t_lo, t_hi, rowi, coli, out_lo, out_hi,
                   idx_r, idx_c, rows_buf, zbuf, shared, sem)

    return spmm


_spmm_n = _make_spmm(N)
_spmm_np = _make_spmm(NP)


BN = 1000
GRID = N // BN


def _mlp_body(x_ref, w1_ref, b1_ref, w2_ref, b2_ref, d0_ref, d1_ref,
              ulo_ref, uhi_ref, h0_ref, dinv_ref):
    h = jnp.maximum(
        jnp.dot(x_ref[...], w1_ref[...], preferred_element_type=jnp.float32)
        + b1_ref[...], 0.0)
    h0 = jnp.dot(h, w2_ref[...], preferred_element_type=jnp.float32) + b2_ref[...]
    deg = d0_ref[...] + d1_ref[...]
    dinv = jnp.where(deg > 0, lax.rsqrt(jnp.where(deg > 0, deg, 1.0)), 0.0)
    u = dinv * h0
    h0_ref[...] = h0
    dinv_ref[...] = dinv
    ulo_ref[...] = u[:, :DPH]
    uhi_ref[...] = jnp.concatenate(
        [u[:, DPH:], jnp.zeros((u.shape[0], DP - HID), jnp.float32)], axis=1)


_mlp_kernel = pl.pallas_call(
    _mlp_body,
    grid=(GRID,),
    in_specs=[
        pl.BlockSpec((BN, D_IN), lambda i: (i, 0)),
        pl.BlockSpec((D_IN, HID), lambda i: (0, 0)),
        pl.BlockSpec((1, HID), lambda i: (0, 0)),
        pl.BlockSpec((HID, HID), lambda i: (0, 0)),
        pl.BlockSpec((1, HID), lambda i: (0, 0)),
        pl.BlockSpec((BN, 1), lambda i: (i, 0)),
        pl.BlockSpec((BN, 1), lambda i: (i, 0)),
    ],
    out_specs=[
        pl.BlockSpec((BN, DPH), lambda i: (i, 0)),
        pl.BlockSpec((BN, DPH), lambda i: (i, 0)),
        pl.BlockSpec((BN, HID), lambda i: (i, 0)),
        pl.BlockSpec((BN, 1), lambda i: (i, 0)),
    ],
    out_shape=[
        jax.ShapeDtypeStruct((N, DPH), jnp.float32),
        jax.ShapeDtypeStruct((N, DPH), jnp.float32),
        jax.ShapeDtypeStruct((N, HID), jnp.float32),
        jax.ShapeDtypeStruct((N, 1), jnp.float32),
    ],
)


def _rescale_body(vlo_ref, vhi_ref, dinv_ref, u2lo_ref, u2hi_ref, p0_ref):
    v = jnp.concatenate([vlo_ref[...], vhi_ref[...]], axis=1)
    dinv = dinv_ref[...]
    p0 = dinv * v
    u2 = dinv * p0
    u2lo_ref[...] = u2[:, :DPH]
    u2hi_ref[...] = u2[:, DPH:]
    p0_ref[...] = p0


_rescale_kernel = pl.pallas_call(
    _rescale_body,
    grid=(GRID,),
    in_specs=[
        pl.BlockSpec((BN, DPH), lambda i: (i, 0)),
        pl.BlockSpec((BN, DPH), lambda i: (i, 0)),
        pl.BlockSpec((BN, 1), lambda i: (i, 0)),
    ],
    out_specs=[
        pl.BlockSpec((BN, DPH), lambda i: (i, 0)),
        pl.BlockSpec((BN, DPH), lambda i: (i, 0)),
        pl.BlockSpec((BN, DP), lambda i: (i, 0)),
    ],
    out_shape=[
        jax.ShapeDtypeStruct((NP, DPH), jnp.float32),
        jax.ShapeDtypeStruct((NP, DPH), jnp.float32),
        jax.ShapeDtypeStruct((N, DP), jnp.float32),
    ],
)


def _head_body(h0_ref, p0_ref, wlo_ref, whi_ref, dinv_ref, th_ref, fb_ref,
               wf_ref, bf_ref, wx_ref, bx_ref, wc_ref, bc_ref, y_ref):
    h0 = h0_ref[...]
    p0 = p0_ref[:, :HID]
    w = jnp.concatenate([wlo_ref[...], whi_ref[...]], axis=1)[:, :HID]
    q = dinv_ref[...] * w
    b0 = h0 + 2.0 * p0 + q
    b1 = h0 - q
    b2 = h0 - 2.0 * p0 + q

    xp = jnp.tanh(
        jnp.dot(h0, wx_ref[...], preferred_element_type=jnp.float32)
        + bx_ref[...])

    cks = (0.25, 0.5, 0.25)
    hs = []
    ls = []
    for f in range(FN):
        t0 = jnp.maximum(th_ref[f, 0], 0.0) * cks[0]
        t1 = jnp.maximum(th_ref[f, 1], 0.0) * cks[1]
        t2 = jnp.maximum(th_ref[f, 2], 0.0) * cks[2]
        hf = t0 * b0 + t1 * b1 + t2 * b2 + fb_ref[f:f + 1, :]
        hp = jnp.tanh(
            jnp.dot(hf, wf_ref[...], preferred_element_type=jnp.float32)
            + bf_ref[...])
        ls.append(jnp.sum(hp * xp, axis=1, keepdims=True))
        hs.append(hf)
    m = jnp.maximum(jnp.maximum(ls[0], ls[1]), jnp.maximum(ls[2], ls[3]))
    es = [jnp.exp(l - m) for l in ls]
    tot = es[0] + es[1] + es[2] + es[3]
    res = (es[0] * hs[0] + es[1] * hs[1] + es[2] * hs[2] + es[3] * hs[3]) / tot
    y_ref[...] = (jnp.dot(res, wc_ref[...], preferred_element_type=jnp.float32)
                  + bc_ref[...])


_head_kernel = pl.pallas_call(
    _head_body,
    grid=(GRID,),
    in_specs=[
        pl.BlockSpec((BN, HID), lambda i: (i, 0)),
        pl.BlockSpec((BN, DP), lambda i: (i, 0)),
        pl.BlockSpec((BN, DPH), lambda i: (i, 0)),
        pl.BlockSpec((BN, DPH), lambda i: (i, 0)),
        pl.BlockSpec((BN, 1), lambda i: (i, 0)),
        pl.BlockSpec(memory_space=pltpu.SMEM),
        pl.BlockSpec((FN, HID), lambda i: (0, 0)),
        pl.BlockSpec((HID, HID), lambda i: (0, 0)),
        pl.BlockSpec((1, HID), lambda i: (0, 0)),
        pl.BlockSpec((HID, HID), lambda i: (0, 0)),
        pl.BlockSpec((1, HID), lambda i: (0, 0)),
        pl.BlockSpec((HID, D_OUT), lambda i: (0, 0)),
        pl.BlockSpec((1, D_OUT), lambda i: (0, 0)),
    ],
    out_specs=pl.BlockSpec((BN, D_OUT), lambda i: (i, 0)),
    out_shape=jax.ShapeDtypeStruct((N, D_OUT), jnp.float32),
)


def kernel(x, edge_index, W1, b1, W2, b2, thetas, fbias, Wf, bf, Wx, bx, Wc, bc):
    row = edge_index[0]
    col = edge_index[1]

    deg0, deg1 = _deg_kernel(row)
    d0 = deg0[:N, 0:1]
    d1 = deg1[:N, 0:1]

    u_lo, u_hi, h0, dinv = _mlp_kernel(
        x, W1, b1.reshape(1, HID), W2, b2.reshape(1, HID), d0, d1)

    v_lo, v_hi = _spmm_n(u_lo, u_hi, row, col)
    u2_lo, u2_hi, p0 = _rescale_kernel(v_lo[:N], v_hi[:N], dinv)
    w_lo, w_hi = _spmm_np(u2_lo, u2_hi, row, col)

    y = _head_kernel(
        h0, p0, w_lo[:N], w_hi[:N], dinv, thetas, fbias,
        Wf, bf.reshape(1, HID), Wx, bx.reshape(1, HID),
        Wc, bc.reshape(1, D_OUT))
    return y


# speedup vs baseline: 16.7100x; 1.9166x over previous
import functools

import jax
import jax.numpy as jnp
from jax import lax
from jax.experimental import pallas as pl
from jax.experimental.pallas import tpu as pltpu
from jax.experimental.pallas import tpu_sc as plsc

N = 10000
E = 320000
D_IN = 128
HID = 156
FN = 4
D_OUT = 2

NC = 2
NS = 16
NP = 10240
DP = 160
DPH = DP // 2
CH = 80
PER_TILE = E // NS
NCH = PER_TILE // CH
PER_TILE_DEG = E // (NC * NS)
NCH_DEG = PER_TILE_DEG // CH
RPT = NP // NS
WCH = 128
NWC = RPT // WCH

_mesh = plsc.VectorSubcoreMesh(
    core_axis_name="c", subcore_axis_name="s", num_cores=NC, num_subcores=NS)
_sc_params = pltpu.CompilerParams(use_tc_tiling_on_sc=False)


def _zero_fill(zbuf, rows, width):
    per_row = width // 16
    zv = jnp.zeros((16,), jnp.float32)

    def body(i, _):
        r = i // per_row
        cix = (i % per_row) * 16
        zbuf[r, pl.ds(cix, 16)] = zv
        return 0

    lax.fori_loop(0, rows * per_row, body, 0)


def _zero_shared(shared, zbuf, sidx):
    def zchunk(j, _):
        pltpu.sync_copy(zbuf, shared.at[pl.ds(sidx * RPT + j * WCH, WCH)])
        return 0

    lax.fori_loop(0, NWC, zchunk, 0)


def _drain_shared(shared, zbuf, out, sidx):
    def wchunk(j, _):
        start = sidx * RPT + j * WCH
        pltpu.sync_copy(shared.at[pl.ds(start, WCH)], zbuf)
        pltpu.sync_copy(zbuf, out.at[pl.ds(start, WCH)])
        return 0

    lax.fori_loop(0, NWC, wchunk, 0)


def _deg_body(rowi2, out0, out1, idx_c, ones_buf, zbuf, shared):
    cidx = lax.axis_index("c")
    sidx = lax.axis_index("s")
    wid = sidx * NC + cidx

    lane = lax.broadcasted_iota(jnp.int32, (16,), 0)
    onehot = jnp.where(lane == 0, 1.0, 0.0).astype(jnp.float32)

    def fill(i, _):
        ones_buf[i, pl.ds(0, 16)] = onehot
        return 0

    lax.fori_loop(0, CH, fill, 0)
    _zero_fill(zbuf, WCH, 16)
    _zero_shared(shared, zbuf, sidx)
    pltpu.sync_copy(rowi2.at[pl.ds(wid * NCH_DEG, NCH_DEG)], idx_c)
    plsc.subcore_barrier()

    def echunk(k, _):
        pltpu.sync_copy(ones_buf, shared.at[idx_c.at[k]], add=True)
        return 0

    lax.fori_loop(0, NCH_DEG, echunk, 0)
    plsc.subcore_barrier()

    @pl.when(cidx == 0)
    def _():
        _drain_shared(shared, zbuf, out0, sidx)

    @pl.when(cidx == 1)
    def _():
        _drain_shared(shared, zbuf, out1, sidx)


_deg_kernel = functools.partial(
    pl.kernel,
    out_type=(
        jax.ShapeDtypeStruct((NP, 16), jnp.float32),
        jax.ShapeDtypeStruct((NP, 16), jnp.float32),
    ),
    mesh=_mesh,
    scratch_types=[
        pltpu.VMEM((NCH_DEG, CH), jnp.int32),
        pltpu.VMEM((CH, 16), jnp.float32),
        pltpu.VMEM((WCH, 16), jnp.float32),
        pltpu.VMEM_SHARED((NP, 16), jnp.float32),
    ],
    compiler_params=_sc_params,
    name="deg_sc",
)(_deg_body)


def _spmm_half(table, out, rowi2, coli2,
               idx_r, idx_c, rows0, rows1, zbuf, shared, sem0, sem1, sidx):
    _zero_fill(zbuf, WCH, DPH)
    _zero_shared(shared, zbuf, sidx)
    pltpu.sync_copy(rowi2.at[pl.ds(sidx * NCH, NCH)], idx_r)
    pltpu.sync_copy(coli2.at[pl.ds(sidx * NCH, NCH)], idx_c)
    plsc.subcore_barrier()

    pltpu.async_copy(table.at[idx_r.at[0]], rows0, sem0)

    def body(j, _):
        k0 = 2 * j
        k1 = k0 + 1
        pltpu.make_async_copy(table.at[idx_r.at[k0]], rows0, sem0).wait()
        pltpu.async_copy(table.at[idx_r.at[k1]], rows1, sem1)
        pltpu.sync_copy(rows0, shared.at[idx_c.at[k0]], add=True)
        pltpu.make_async_copy(table.at[idx_r.at[k1]], rows1, sem1).wait()

        @pl.when(j < NCH // 2 - 1)
        def _():
            pltpu.async_copy(table.at[idx_r.at[k0 + 2]], rows0, sem0)

        pltpu.sync_copy(rows1, shared.at[idx_c.at[k1]], add=True)
        return 0

    lax.fori_loop(0, NCH // 2, body, 0)
    plsc.subcore_barrier()
    _drain_shared(shared, zbuf, out, sidx)


def _spmm_body(t_lo, t_hi, rowi2, coli2, out_lo, out_hi,
               idx_r, idx_c, rows0, rows1, zbuf, shared, sem0, sem1):
    cidx = lax.axis_index("c")
    sidx = lax.axis_index("s")

    @pl.when(cidx == 0)
    def _():
        _spmm_half(t_lo, out_lo, rowi2, coli2,
                   idx_r, idx_c, rows0, rows1, zbuf, shared, sem0, sem1, sidx)

    @pl.when(cidx == 1)
    def _():
        _spmm_half(t_hi, out_hi, rowi2, coli2,
                   idx_r, idx_c, rows0, rows1, zbuf, shared, sem0, sem1, sidx)


def _make_spmm(nv):
    @functools.partial(
        pl.kernel,
        out_type=(
            jax.ShapeDtypeStruct((NP, DPH), jnp.float32),
            jax.ShapeDtypeStruct((NP, DPH), jnp.float32),
        ),
        mesh=_mesh,
        scratch_types=[
            pltpu.VMEM((NCH, CH), jnp.int32),
            pltpu.VMEM((NCH, CH), jnp.int32),
            pltpu.VMEM((CH, DPH), jnp.float32),
            pltpu.VMEM((CH, DPH), jnp.float32),
            pltpu.VMEM((WCH, DPH), jnp.float32),
            pltpu.VMEM_SHARED((NP, DPH), jnp.float32),
            pltpu.SemaphoreType.DMA,
            pltpu.SemaphoreType.DMA,
        ],
        compiler_params=_sc_params,
        name=f"spmm_sc_{nv}",
    )
    def spmm(t_lo, t_hi, rowi2, coli2, out_lo, out_hi,
             idx_r, idx_c, rows0, rows1, zbuf, shared, sem0, sem1):
        _spmm_body(t_lo, t_hi, rowi2, coli2, out_lo, out_hi,
                   idx_r, idx_c, rows0, rows1, zbuf, shared, sem0, sem1)

    return spmm


_spmm_n = _make_spmm(N)
_spmm_np = _make_spmm(NP)


BN = 1000
GRID = N // BN


def _mlp_body(x_ref, w1_ref, b1_ref, w2_ref, b2_ref, d0_ref, d1_ref,
              ulo_ref, uhi_ref, h0_ref, dinv_ref):
    h = jnp.maximum(
        jnp.dot(x_ref[...], w1_ref[...], preferred_element_type=jnp.float32)
        + b1_ref[...], 0.0)
    h0 = jnp.dot(h, w2_ref[...], preferred_element_type=jnp.float32) + b2_ref[...]
    deg = d0_ref[...] + d1_ref[...]
    dinv = jnp.where(deg > 0, lax.rsqrt(jnp.where(deg > 0, deg, 1.0)), 0.0)
    u = dinv * h0
    h0_ref[...] = h0
    dinv_ref[...] = dinv
    ulo_ref[...] = u[:, :DPH]
    uhi_ref[...] = jnp.concatenate(
        [u[:, DPH:], jnp.zeros((u.shape[0], DP - HID), jnp.float32)], axis=1)


_mlp_kernel = pl.pallas_call(
    _mlp_body,
    grid=(GRID,),
    in_specs=[
        pl.BlockSpec((BN, D_IN), lambda i: (i, 0)),
        pl.BlockSpec((D_IN, HID), lambda i: (0, 0)),
        pl.BlockSpec((1, HID), lambda i: (0, 0)),
        pl.BlockSpec((HID, HID), lambda i: (0, 0)),
        pl.BlockSpec((1, HID), lambda i: (0, 0)),
        pl.BlockSpec((BN, 1), lambda i: (i, 0)),
        pl.BlockSpec((BN, 1), lambda i: (i, 0)),
    ],
    out_specs=[
        pl.BlockSpec((BN, DPH), lambda i: (i, 0)),
        pl.BlockSpec((BN, DPH), lambda i: (i, 0)),
        pl.BlockSpec((BN, HID), lambda i: (i, 0)),
        pl.BlockSpec((BN, 1), lambda i: (i, 0)),
    ],
    out_shape=[
        jax.ShapeDtypeStruct((N, DPH), jnp.float32),
        jax.ShapeDtypeStruct((N, DPH), jnp.float32),
        jax.ShapeDtypeStruct((N, HID), jnp.float32),
        jax.ShapeDtypeStruct((N, 1), jnp.float32),
    ],
)


def _rescale_body(vlo_ref, vhi_ref, dinv_ref, u2lo_ref, u2hi_ref, p0_ref):
    v = jnp.concatenate([vlo_ref[...], vhi_ref[...]], axis=1)
    dinv = dinv_ref[...]
    p0 = dinv * v
    u2 = dinv * p0
    u2lo_ref[...] = u2[:, :DPH]
    u2hi_ref[...] = u2[:, DPH:]
    p0_ref[...] = p0


_rescale_kernel = pl.pallas_call(
    _rescale_body,
    grid=(GRID,),
    in_specs=[
        pl.BlockSpec((BN, DPH), lambda i: (i, 0)),
        pl.BlockSpec((BN, DPH), lambda i: (i, 0)),
        pl.BlockSpec((BN, 1), lambda i: (i, 0)),
    ],
    out_specs=[
        pl.BlockSpec((BN, DPH), lambda i: (i, 0)),
        pl.BlockSpec((BN, DPH), lambda i: (i, 0)),
        pl.BlockSpec((BN, DP), lambda i: (i, 0)),
    ],
    out_shape=[
        jax.ShapeDtypeStruct((NP, DPH), jnp.float32),
        jax.ShapeDtypeStruct((NP, DPH), jnp.float32),
        jax.ShapeDtypeStruct((N, DP), jnp.float32),
    ],
)


def _head_body(h0_ref, p0_ref, wlo_ref, whi_ref, dinv_ref, th_ref, fb_ref,
               wf_ref, bf_ref, wx_ref, bx_ref, wc_ref, bc_ref, y_ref):
    h0 = h0_ref[...]
    p0 = p0_ref[:, :HID]
    w = jnp.concatenate([wlo_ref[...], whi_ref[...]], axis=1)[:, :HID]
    q = dinv_ref[...] * w
    b0 = h0 + 2.0 * p0 + q
    b1 = h0 - q
    b2 = h0 - 2.0 * p0 + q

    xp = jnp.tanh(
        jnp.dot(h0, wx_ref[...], preferred_element_type=jnp.float32)
        + bx_ref[...])

    cks = (0.25, 0.5, 0.25)
    hs = []
    ls = []
    for f in range(FN):
        t0 = jnp.maximum(th_ref[f, 0], 0.0) * cks[0]
        t1 = jnp.maximum(th_ref[f, 1], 0.0) * cks[1]
        t2 = jnp.maximum(th_ref[f, 2], 0.0) * cks[2]
        hf = t0 * b0 + t1 * b1 + t2 * b2 + fb_ref[f:f + 1, :]
        hp = jnp.tanh(
            jnp.dot(hf, wf_ref[...], preferred_element_type=jnp.float32)
            + bf_ref[...])
        ls.append(jnp.sum(hp * xp, axis=1, keepdims=True))
        hs.append(hf)
    m = jnp.maximum(jnp.maximum(ls[0], ls[1]), jnp.maximum(ls[2], ls[3]))
    es = [jnp.exp(l - m) for l in ls]
    tot = es[0] + es[1] + es[2] + es[3]
    res = (es[0] * hs[0] + es[1] * hs[1] + es[2] * hs[2] + es[3] * hs[3]) / tot
    y_ref[...] = (jnp.dot(res, wc_ref[...], preferred_element_type=jnp.float32)
                  + bc_ref[...])


_head_kernel = pl.pallas_call(
    _head_body,
    grid=(GRID,),
    in_specs=[
        pl.BlockSpec((BN, HID), lambda i: (i, 0)),
        pl.BlockSpec((BN, DP), lambda i: (i, 0)),
        pl.BlockSpec((BN, DPH), lambda i: (i, 0)),
        pl.BlockSpec((BN, DPH), lambda i: (i, 0)),
        pl.BlockSpec((BN, 1), lambda i: (i, 0)),
        pl.BlockSpec(memory_space=pltpu.SMEM),
        pl.BlockSpec((FN, HID), lambda i: (0, 0)),
        pl.BlockSpec((HID, HID), lambda i: (0, 0)),
        pl.BlockSpec((1, HID), lambda i: (0, 0)),
        pl.BlockSpec((HID, HID), lambda i: (0, 0)),
        pl.BlockSpec((1, HID), lambda i: (0, 0)),
        pl.BlockSpec((HID, D_OUT), lambda i: (0, 0)),
        pl.BlockSpec((1, D_OUT), lambda i: (0, 0)),
    ],
    out_specs=pl.BlockSpec((BN, D_OUT), lambda i: (i, 0)),
    out_shape=jax.ShapeDtypeStruct((N, D_OUT), jnp.float32),
)


def kernel(x, edge_index, W1, b1, W2, b2, thetas, fbias, Wf, bf, Wx, bx, Wc, bc):
    row2 = edge_index[0].reshape(E // CH, CH)
    col2 = edge_index[1].reshape(E // CH, CH)

    deg0, deg1 = _deg_kernel(row2)
    d0 = deg0[:N, 0:1]
    d1 = deg1[:N, 0:1]

    u_lo, u_hi, h0, dinv = _mlp_kernel(
        x, W1, b1.reshape(1, HID), W2, b2.reshape(1, HID), d0, d1)

    v_lo, v_hi = _spmm_n(u_lo, u_hi, row2, col2)
    u2_lo, u2_hi, p0 = _rescale_kernel(v_lo[:N], v_hi[:N], dinv)
    w_lo, w_hi = _spmm_np(u2_lo, u2_hi, row2, col2)

    y = _head_kernel(
        h0, p0, w_lo[:N], w_hi[:N], dinv, thetas, fbias,
        Wf, bf.reshape(1, HID), Wx, bx.reshape(1, HID),
        Wc, bc.reshape(1, D_OUT))
    return y


# speedup vs baseline: 26.0089x; 1.5565x over previous
import functools

import jax
import jax.numpy as jnp
from jax import lax
from jax.experimental import pallas as pl
from jax.experimental.pallas import tpu as pltpu
from jax.experimental.pallas import tpu_sc as plsc

N = 10000
E = 320000
D_IN = 128
HID = 156
FN = 4
D_OUT = 2

NC = 2
NS = 16
NP = 10240
DP = 160
DPH = DP // 2
CH = 80
PER_TILE = E // NS
NCH = PER_TILE // CH
PER_TILE_DEG = E // (NC * NS)
NCH_DEG = PER_TILE_DEG // CH
RPT = NP // NS
WCH = 64
NWC = RPT // WCH
NB = 5

_mesh = plsc.VectorSubcoreMesh(
    core_axis_name="c", subcore_axis_name="s", num_cores=NC, num_subcores=NS)
_sc_params = pltpu.CompilerParams(use_tc_tiling_on_sc=False)


def _zero_fill(zbuf, rows, width):
    per_row = width // 16
    zv = jnp.zeros((16,), jnp.float32)

    def body(i, _):
        r = i // per_row
        cix = (i % per_row) * 16
        zbuf[r, pl.ds(cix, 16)] = zv
        return 0

    lax.fori_loop(0, rows * per_row, body, 0)


def _zero_shared(shared, zbuf, sidx):
    def zchunk(j, _):
        pltpu.sync_copy(zbuf, shared.at[pl.ds(sidx * RPT + j * WCH, WCH)])
        return 0

    lax.fori_loop(0, NWC, zchunk, 0)


def _drain_shared(shared, zbuf, out, sidx):
    def wchunk(j, _):
        start = sidx * RPT + j * WCH
        pltpu.sync_copy(shared.at[pl.ds(start, WCH)], zbuf)
        pltpu.sync_copy(zbuf, out.at[pl.ds(start, WCH)])
        return 0

    lax.fori_loop(0, NWC, wchunk, 0)


def _deg_body(rowi2, out0, out1, idx_c, ones_buf, zbuf, shared, dsem):
    cidx = lax.axis_index("c")
    sidx = lax.axis_index("s")
    wid = sidx * NC + cidx

    lane = lax.broadcasted_iota(jnp.int32, (16,), 0)
    onehot = jnp.where(lane == 0, 1.0, 0.0).astype(jnp.float32)

    def fill(i, _):
        ones_buf[i, pl.ds(0, 16)] = onehot
        return 0

    lax.fori_loop(0, CH, fill, 0)
    _zero_fill(zbuf, WCH, 16)
    _zero_shared(shared, zbuf, sidx)
    pltpu.sync_copy(rowi2.at[pl.ds(wid * NCH_DEG, NCH_DEG)], idx_c)
    plsc.subcore_barrier()

    def echunk(k, _):
        @pl.when(k >= 8)
        def _():
            pltpu.make_async_copy(
                ones_buf, shared.at[idx_c.at[0]], dsem).wait()

        pltpu.async_copy(ones_buf, shared.at[idx_c.at[k]], dsem, add=True)
        return 0

    lax.fori_loop(0, NCH_DEG, echunk, 0)

    def edrain(k, _):
        pltpu.make_async_copy(ones_buf, shared.at[idx_c.at[0]], dsem).wait()
        return 0

    lax.fori_loop(0, 8, edrain, 0)
    plsc.subcore_barrier()

    @pl.when(cidx == 0)
    def _():
        _drain_shared(shared, zbuf, out0, sidx)

    @pl.when(cidx == 1)
    def _():
        _drain_shared(shared, zbuf, out1, sidx)


_deg_kernel = functools.partial(
    pl.kernel,
    out_type=(
        jax.ShapeDtypeStruct((NP, 16), jnp.float32),
        jax.ShapeDtypeStruct((NP, 16), jnp.float32),
    ),
    mesh=_mesh,
    scratch_types=[
        pltpu.VMEM((NCH_DEG, CH), jnp.int32),
        pltpu.VMEM((CH, 16), jnp.float32),
        pltpu.VMEM((WCH, 16), jnp.float32),
        pltpu.VMEM_SHARED((NP, 16), jnp.float32),
        pltpu.SemaphoreType.DMA,
    ],
    compiler_params=_sc_params,
    name="deg_sc",
)(_deg_body)


def _spmm_half(table, out, rowi2, coli2,
               idx_r, idx_c, rows_bufs, zbuf, shared,
               gather_sems, scatter_sems, sidx):
    _zero_fill(zbuf, WCH, DPH)
    _zero_shared(shared, zbuf, sidx)
    pltpu.sync_copy(rowi2.at[pl.ds(sidx * NCH, NCH)], idx_r)
    pltpu.sync_copy(coli2.at[pl.ds(sidx * NCH, NCH)], idx_c)
    plsc.subcore_barrier()

    rbufs = list(rows_bufs)
    gsems = list(gather_sems)
    ssems = list(scatter_sems)
    rounds = NCH // NB

    for b in range(NB):
        pltpu.async_copy(table.at[idx_r.at[b]], rbufs[b], gsems[b])

    def round_body(g, _):
        base = g * NB
        for b in range(NB):
            pltpu.make_async_copy(
                table.at[idx_r.at[base + b]], rbufs[b], gsems[b]).wait()
            pltpu.async_copy(
                rbufs[b], shared.at[idx_c.at[base + b]], ssems[b], add=True)
        for b in range(NB):
            pltpu.make_async_copy(
                rbufs[b], shared.at[idx_c.at[base + b]], ssems[b]).wait()

            @pl.when(g < rounds - 1)
            def _(b=b):
                pltpu.async_copy(
                    table.at[idx_r.at[base + NB + b]], rbufs[b], gsems[b])

        return 0

    lax.fori_loop(0, rounds, round_body, 0)
    plsc.subcore_barrier()
    _drain_shared(shared, zbuf, out, sidx)


def _spmm_body(t_lo, t_hi, rowi2, coli2, out_lo, out_hi,
               idx_r, idx_c, rows_bufs, zbuf, shared,
               gather_sems, scatter_sems):
    cidx = lax.axis_index("c")
    sidx = lax.axis_index("s")

    @pl.when(cidx == 0)
    def _():
        _spmm_half(t_lo, out_lo, rowi2, coli2,
                   idx_r, idx_c, rows_bufs, zbuf, shared,
                   gather_sems, scatter_sems, sidx)

    @pl.when(cidx == 1)
    def _():
        _spmm_half(t_hi, out_hi, rowi2, coli2,
                   idx_r, idx_c, rows_bufs, zbuf, shared,
                   gather_sems, scatter_sems, sidx)


def _make_spmm(nv):
    @functools.partial(
        pl.kernel,
        out_type=(
            jax.ShapeDtypeStruct((NP, DPH), jnp.float32),
            jax.ShapeDtypeStruct((NP, DPH), jnp.float32),
        ),
        mesh=_mesh,
        scratch_types=[
            pltpu.VMEM((NCH, CH), jnp.int32),
            pltpu.VMEM((NCH, CH), jnp.int32),
            [pltpu.VMEM((CH, DPH), jnp.float32) for _ in range(NB)],
            pltpu.VMEM((WCH, DPH), jnp.float32),
            pltpu.VMEM_SHARED((NP, DPH), jnp.float32),
            [pltpu.SemaphoreType.DMA for _ in range(NB)],
            [pltpu.SemaphoreType.DMA for _ in range(NB)],
        ],
        compiler_params=_sc_params,
        name=f"spmm_sc_{nv}",
    )
    def spmm(t_lo, t_hi, rowi2, coli2, out_lo, out_hi,
             idx_r, idx_c, rows_bufs, zbuf, shared, gather_sems, scatter_sems):
        _spmm_body(t_lo, t_hi, rowi2, coli2, out_lo, out_hi,
                   idx_r, idx_c, rows_bufs, zbuf, shared,
                   gather_sems, scatter_sems)

    return spmm


_spmm_n = _make_spmm(N)
_spmm_np = _make_spmm(NP)


BN = 1000
GRID = N // BN


def _mlp_body(x_ref, w1_ref, b1_ref, w2_ref, b2_ref, d0_ref, d1_ref,
              ulo_ref, uhi_ref, h0_ref, dinv_ref):
    h = jnp.maximum(
        jnp.dot(x_ref[...], w1_ref[...], preferred_element_type=jnp.float32)
        + b1_ref[...], 0.0)
    h0 = jnp.dot(h, w2_ref[...], preferred_element_type=jnp.float32) + b2_ref[...]
    deg = d0_ref[:, :1] + d1_ref[:, :1]
    dinv = jnp.where(deg > 0, lax.rsqrt(jnp.where(deg > 0, deg, 1.0)), 0.0)
    u = dinv * h0
    h0_ref[...] = h0
    dinv_ref[...] = dinv
    ulo_ref[...] = u[:, :DPH]
    uhi_ref[...] = jnp.concatenate(
        [u[:, DPH:], jnp.zeros((u.shape[0], DP - HID), jnp.float32)], axis=1)


_mlp_kernel = pl.pallas_call(
    _mlp_body,
    grid=(GRID,),
    in_specs=[
        pl.BlockSpec((BN, D_IN), lambda i: (i, 0)),
        pl.BlockSpec((D_IN, HID), lambda i: (0, 0)),
        pl.BlockSpec((1, HID), lambda i: (0, 0)),
        pl.BlockSpec((HID, HID), lambda i: (0, 0)),
        pl.BlockSpec((1, HID), lambda i: (0, 0)),
        pl.BlockSpec((BN, 16), lambda i: (i, 0)),
        pl.BlockSpec((BN, 16), lambda i: (i, 0)),
    ],
    out_specs=[
        pl.BlockSpec((BN, DPH), lambda i: (i, 0)),
        pl.BlockSpec((BN, DPH), lambda i: (i, 0)),
        pl.BlockSpec((BN, HID), lambda i: (i, 0)),
        pl.BlockSpec((BN, 1), lambda i: (i, 0)),
    ],
    out_shape=[
        jax.ShapeDtypeStruct((N, DPH), jnp.float32),
        jax.ShapeDtypeStruct((N, DPH), jnp.float32),
        jax.ShapeDtypeStruct((N, HID), jnp.float32),
        jax.ShapeDtypeStruct((N, 1), jnp.float32),
    ],
)


def _rescale_body(vlo_ref, vhi_ref, dinv_ref, u2lo_ref, u2hi_ref, p0_ref):
    v = jnp.concatenate([vlo_ref[...], vhi_ref[...]], axis=1)
    dinv = dinv_ref[...]
    p0 = dinv * v
    u2 = dinv * p0
    u2lo_ref[...] = u2[:, :DPH]
    u2hi_ref[...] = u2[:, DPH:]
    p0_ref[...] = p0


_rescale_kernel = pl.pallas_call(
    _rescale_body,
    grid=(GRID,),
    in_specs=[
        pl.BlockSpec((BN, DPH), lambda i: (i, 0)),
        pl.BlockSpec((BN, DPH), lambda i: (i, 0)),
        pl.BlockSpec((BN, 1), lambda i: (i, 0)),
    ],
    out_specs=[
        pl.BlockSpec((BN, DPH), lambda i: (i, 0)),
        pl.BlockSpec((BN, DPH), lambda i: (i, 0)),
        pl.BlockSpec((BN, DP), lambda i: (i, 0)),
    ],
    out_shape=[
        jax.ShapeDtypeStruct((NP, DPH), jnp.float32),
        jax.ShapeDtypeStruct((NP, DPH), jnp.float32),
        jax.ShapeDtypeStruct((N, DP), jnp.float32),
    ],
)


def _head_body(h0_ref, p0_ref, wlo_ref, whi_ref, dinv_ref, th_ref, fb_ref,
               wf_ref, bf_ref, wx_ref, bx_ref, wc_ref, bc_ref, y_ref):
    h0 = h0_ref[...]
    p0 = p0_ref[:, :HID]
    w = jnp.concatenate([wlo_ref[...], whi_ref[...]], axis=1)[:, :HID]
    q = dinv_ref[...] * w
    b0 = h0 + 2.0 * p0 + q
    b1 = h0 - q
    b2 = h0 - 2.0 * p0 + q

    xp = jnp.tanh(
        jnp.dot(h0, wx_ref[...], preferred_element_type=jnp.float32)
        + bx_ref[...])

    cks = (0.25, 0.5, 0.25)
    hs = []
    ls = []
    for f in range(FN):
        t0 = jnp.maximum(th_ref[f, 0], 0.0) * cks[0]
        t1 = jnp.maximum(th_ref[f, 1], 0.0) * cks[1]
        t2 = jnp.maximum(th_ref[f, 2], 0.0) * cks[2]
        hf = t0 * b0 + t1 * b1 + t2 * b2 + fb_ref[f:f + 1, :]
        hp = jnp.tanh(
            jnp.dot(hf, wf_ref[...], preferred_element_type=jnp.float32)
            + bf_ref[...])
        ls.append(jnp.sum(hp * xp, axis=1, keepdims=True))
        hs.append(hf)
    m = jnp.maximum(jnp.maximum(ls[0], ls[1]), jnp.maximum(ls[2], ls[3]))
    es = [jnp.exp(l - m) for l in ls]
    tot = es[0] + es[1] + es[2] + es[3]
    res = (es[0] * hs[0] + es[1] * hs[1] + es[2] * hs[2] + es[3] * hs[3]) / tot
    y_ref[...] = (jnp.dot(res, wc_ref[...], preferred_element_type=jnp.float32)
                  + bc_ref[...])


_head_kernel = pl.pallas_call(
    _head_body,
    grid=(GRID,),
    in_specs=[
        pl.BlockSpec((BN, HID), lambda i: (i, 0)),
        pl.BlockSpec((BN, DP), lambda i: (i, 0)),
        pl.BlockSpec((BN, DPH), lambda i: (i, 0)),
        pl.BlockSpec((BN, DPH), lambda i: (i, 0)),
        pl.BlockSpec((BN, 1), lambda i: (i, 0)),
        pl.BlockSpec(memory_space=pltpu.SMEM),
        pl.BlockSpec((FN, HID), lambda i: (0, 0)),
        pl.BlockSpec((HID, HID), lambda i: (0, 0)),
        pl.BlockSpec((1, HID), lambda i: (0, 0)),
        pl.BlockSpec((HID, HID), lambda i: (0, 0)),
        pl.BlockSpec((1, HID), lambda i: (0, 0)),
        pl.BlockSpec((HID, D_OUT), lambda i: (0, 0)),
        pl.BlockSpec((1, D_OUT), lambda i: (0, 0)),
    ],
    out_specs=pl.BlockSpec((BN, D_OUT), lambda i: (i, 0)),
    out_shape=jax.ShapeDtypeStruct((N, D_OUT), jnp.float32),
)


def kernel(x, edge_index, W1, b1, W2, b2, thetas, fbias, Wf, bf, Wx, bx, Wc, bc):
    row2 = edge_index[0].reshape(E // CH, CH)
    col2 = edge_index[1].reshape(E // CH, CH)

    deg0, deg1 = _deg_kernel(row2)

    u_lo, u_hi, h0, dinv = _mlp_kernel(
        x, W1, b1.reshape(1, HID), W2, b2.reshape(1, HID), deg0, deg1)

    v_lo, v_hi = _spmm_n(u_lo, u_hi, row2, col2)
    u2_lo, u2_hi, p0 = _rescale_kernel(v_lo, v_hi, dinv)
    w_lo, w_hi = _spmm_np(u2_lo, u2_hi, row2, col2)

    y = _head_kernel(
        h0, p0, w_lo, w_hi, dinv, thetas, fbias,
        Wf, bf.reshape(1, HID), Wx, bx.reshape(1, HID),
        Wc, bc.reshape(1, D_OUT))
    return y
